# baseline (speedup 1.0000x reference)
import functools

import jax
import jax.numpy as jnp
from jax import lax
from jax.experimental import pallas as pl
from jax.experimental.pallas import tpu as pltpu
from jax.experimental.pallas import tpu_sc as plsc

N_ATOMS = 2_000_000
N_SPECIES = 119
CHUNK = 8000
N_CHUNKS = N_ATOMS // CHUNK
LANES = 16
UNROLL = 4
NC, NS = 2, 16
NW = NC * NS
ROUNDS = -(-N_CHUNKS // NW)
REM = N_CHUNKS - (ROUNDS - 1) * NW


@functools.cache
def _make_sc_kernel():
    mesh = plsc.VectorSubcoreMesh(
        core_axis_name="c", subcore_axis_name="s", num_cores=NC)

    @functools.partial(
        pl.kernel,
        mesh=mesh,
        out_type=jax.ShapeDtypeStruct((1, N_ATOMS), jnp.float32),
        compiler_params=pltpu.CompilerParams(
            needs_layout_passes=False, disable_bounds_checks=True,
            use_tc_tiling_on_sc=False),
        scratch_types=[
            pltpu.VMEM((N_SPECIES,), jnp.float32),
            pltpu.VMEM((N_SPECIES,), jnp.float32),
            pltpu.VMEM((CHUNK,), jnp.float32),
            pltpu.VMEM((CHUNK,), jnp.float32),
            pltpu.VMEM((CHUNK,), jnp.int32),
            pltpu.VMEM((CHUNK,), jnp.int32),
            pltpu.VMEM((CHUNK,), jnp.float32),
            pltpu.VMEM((CHUNK,), jnp.float32),
            pltpu.SemaphoreType.DMA,
            pltpu.SemaphoreType.DMA,
            pltpu.SemaphoreType.DMA,
            pltpu.SemaphoreType.DMA,
        ],
    )
    def sc_kernel(x_hbm, z_hbm, scale_hbm, shift_hbm, out_hbm,
                  sc_v, sh_v, x_v0, x_v1, z_v0, z_v1, o_v0, o_v1,
                  sem_in0, sem_in1, sem_out0, sem_out1):
        x_v = (x_v0, x_v1)
        z_v = (z_v0, z_v1)
        o_v = (o_v0, o_v1)
        sem_in = (sem_in0, sem_in1)
        sem_out = (sem_out0, sem_out1)
        wid = lax.axis_index("s") * NC + lax.axis_index("c")

        def issue_in(j, slot):
            base = (wid + j * NW) * CHUNK
            pltpu.async_copy(
                z_hbm.at[pl.ds(base, CHUNK)], z_v[slot], sem_in[slot])
            pltpu.async_copy(
                x_hbm.at[0, pl.ds(base, CHUNK)], x_v[slot], sem_in[slot])

        def wait_in(slot):
            pltpu.make_async_copy(
                z_hbm.at[pl.ds(0, CHUNK)], z_v[slot], sem_in[slot]).wait()
            pltpu.make_async_copy(
                x_hbm.at[0, pl.ds(0, CHUNK)], x_v[slot], sem_in[slot]).wait()

        def issue_out(j, slot):
            base = (wid + j * NW) * CHUNK
            pltpu.async_copy(
                o_v[slot], out_hbm.at[0, pl.ds(base, CHUNK)], sem_out[slot])

        def wait_out(slot):
            pltpu.make_async_copy(
                o_v[slot], out_hbm.at[0, pl.ds(0, CHUNK)],
                sem_out[slot]).wait()

        def compute(slot):
            @plsc.parallel_loop(0, CHUNK, step=LANES, unroll=UNROLL)
            def body(i):
                sl = pl.ds(i, LANES)
                zv = z_v[slot][sl]
                xv = x_v[slot][sl]
                sv = plsc.load_gather(sc_v, [zv])
                bv = plsc.load_gather(sh_v, [zv])
                o_v[slot][sl] = sv * xv + bv

        pltpu.sync_copy(scale_hbm.at[0], sc_v)
        pltpu.sync_copy(shift_hbm.at[0], sh_v)

        issue_in(0, 0)
        for j in range(ROUNDS):
            slot = j & 1
            nxt = j + 1
            if nxt < ROUNDS:
                if nxt == ROUNDS - 1:
                    @pl.when(wid < REM)
                    def _():
                        issue_in(nxt, nxt & 1)
                else:
                    issue_in(nxt, nxt & 1)

            def step(j=j, slot=slot):
                wait_in(slot)
                if j >= 2:
                    wait_out(slot)
                compute(slot)
                issue_out(j, slot)

            if j == ROUNDS - 1:
                pl.when(wid < REM)(step)
            else:
                step()

        wait_out(0)
        wait_out(1)

    return sc_kernel


@jax.jit
def kernel(x, Z, scale, shift):
    out = _make_sc_kernel()(x.T, Z.astype(jnp.int32), scale.T, shift.T)
    return out.T
